# baseline (speedup 1.0000x reference)
import functools
import math

import jax
import jax.numpy as jnp
from jax import lax
from jax.experimental import pallas as pl
from jax.experimental.pallas import tpu as pltpu
from jax.experimental.pallas import tpu_sc as plsc


@functools.lru_cache(maxsize=None)
def _build(B, T, D, V):
    info = plsc.get_sparse_core_info()
    NC, NS, L = info.num_cores, info.num_subcores, info.num_lanes
    NW = NC * NS
    BT = 128
    TG = 8
    assert B % (NW * BT) == 0 and T % TG == 0 and D % L == 0
    NBT = B // BT
    cpw = NBT // NW
    ntg = T // TG
    scale = float(math.sqrt(D))

    mesh = plsc.VectorSubcoreMesh(core_axis_name="c", subcore_axis_name="s")

    @functools.partial(
        pl.kernel,
        out_type=jax.ShapeDtypeStruct((T, D // 8, NBT, 8, BT), jnp.float32),
        mesh=mesh,
        compiler_params=pltpu.CompilerParams(use_tc_tiling_on_sc=False,
                                             needs_layout_passes=False),
        scratch_types=[
            pltpu.VMEM((T, D), jnp.float32),
            pltpu.VMEM((ntg, TG, BT), jnp.int32),
            pltpu.VMEM((TG, BT, D), jnp.float32),
            pltpu.VMEM((2, D // 8, 8, BT), jnp.float32),
            pltpu.SemaphoreType.DMA,
            pltpu.SemaphoreType.DMA,
            pltpu.SemaphoreType.DMA,
        ],
    )
    def launch(x4, tok_hbm, pos_hbm, o5, posblk, xc, rowb, outt,
               xsem, gsem, osem):
        wid = lax.axis_index("s") * NC + lax.axis_index("c")
        c0 = wid * cpw
        pltpu.sync_copy(pos_hbm.at[pl.ds(0, T)], posblk)

        for cl in range(cpw):
            c = c0 + cl
            pltpu.async_copy(x4.at[:, c], xc, xsem).wait()

            def tgbody(tg, carry):
                t0 = tg * TG
                gathers = [
                    pltpu.async_copy(tok_hbm.at[xc.at[tg, tt]],
                                     rowb.at[tt], gsem)
                    for tt in range(TG)
                ]
                oput = [None, None]
                for tt in range(TG):
                    t = t0 + tt
                    par = tt % 2
                    gathers[tt].wait()
                    if oput[par] is not None:
                        oput[par].wait()

                    def dbody(d, c2, tt=tt, par=par, t=t):
                        j = d // 8
                        dd = d % 8
                        pv = plsc.load_gather(
                            posblk, [jnp.full((L,), t, jnp.int32),
                                     jnp.full((L,), d, jnp.int32)])
                        tts = jnp.full((L,), tt, jnp.int32)
                        ds_ = jnp.full((L,), d, jnp.int32)
                        for m in range(BT // L):
                            bs = lax.iota(jnp.int32, L) + (m * L)
                            v = plsc.load_gather(rowb, [tts, bs, ds_])
                            outt[par, j, dd, pl.ds(m * L, L)] = (
                                v * scale + pv)
                        return c2

                    lax.fori_loop(0, D, dbody, 0)
                    oput[par] = pltpu.async_copy(
                        outt.at[par], o5.at[t, :, c], osem)
                for h in oput:
                    h.wait()
                return carry

            lax.fori_loop(0, ntg, tgbody, 0)

    return launch


def kernel(x, token_table, pos_table):
    B, T = x.shape
    V, D = token_table.shape
    launch = _build(B, T, D, V)
    x4 = (x.astype(jnp.int32).T
          .reshape(T // 8, 8, B // 128, 128)
          .transpose(0, 2, 1, 3))
    o5 = launch(x4, token_table, pos_table)
    return o5.transpose(2, 4, 0, 1, 3).reshape(B, T, D)


# speedup vs baseline: 1.2646x; 1.2646x over previous
import functools
import math

import jax
import jax.numpy as jnp
from jax import lax
from jax.experimental import pallas as pl
from jax.experimental.pallas import tpu as pltpu
from jax.experimental.pallas import tpu_sc as plsc


@functools.lru_cache(maxsize=None)
def _build(B, T, D, V):
    info = plsc.get_sparse_core_info()
    NC, NS, L = info.num_cores, info.num_subcores, info.num_lanes
    NW = NC * NS
    BT = 128
    TG = 8
    RB = 6
    DP = D + 8
    assert B % (NW * BT) == 0 and T % TG == 0 and D % L == 0
    NBT = B // BT
    cpw = NBT // NW
    ntg = T // TG
    scale = float(math.sqrt(D))

    mesh = plsc.VectorSubcoreMesh(core_axis_name="c", subcore_axis_name="s")

    @functools.partial(
        pl.kernel,
        out_type=jax.ShapeDtypeStruct((T, D // 8, NBT, 8, BT), jnp.float32),
        mesh=mesh,
        compiler_params=pltpu.CompilerParams(use_tc_tiling_on_sc=False,
                                             needs_layout_passes=False),
        scratch_types=[
            pltpu.VMEM((T, D), jnp.float32),
            pltpu.VMEM((ntg, TG, BT), jnp.int32),
            pltpu.VMEM((RB, BT, D), jnp.float32),
            pltpu.VMEM((BT, DP), jnp.float32),
            pltpu.VMEM((2, D // 8, 8, BT), jnp.float32),
            pltpu.SemaphoreType.DMA,
            pltpu.SemaphoreType.DMA,
            pltpu.SemaphoreType.DMA,
        ],
    )
    def launch(x4, tok_hbm, pos_hbm, o5, posblk, xc, rowb, rowp, outt,
               xsem, gsem, osem):
        wid = lax.axis_index("s") * NC + lax.axis_index("c")
        c0 = wid * cpw
        pltpu.sync_copy(pos_hbm.at[pl.ds(0, T)], posblk)

        for cl in range(cpw):
            c = c0 + cl
            pltpu.async_copy(x4.at[:, c], xc, xsem).wait()

            def tgbody(tg, carry):
                t0 = tg * TG
                gathers = [None] * TG
                for tt in range(RB):
                    gathers[tt] = pltpu.async_copy(
                        tok_hbm.at[xc.at[tg, tt]], rowb.at[tt % RB], gsem)
                oput = [None, None]
                for tt in range(TG):
                    t = t0 + tt
                    par = tt % 2
                    gathers[tt].wait()

                    pv = [posblk[t, pl.ds(k * L, L)] for k in range(D // L)]

                    def p1body(b, c2, tt=tt, pv=pv):
                        for k in range(D // L):
                            v = rowb[tt % RB, b, pl.ds(k * L, L)]
                            rowp[b, pl.ds(k * L, L)] = v * scale + pv[k]
                        return c2

                    lax.fori_loop(0, BT, p1body, 0, unroll=2)
                    if tt + RB < TG:
                        gathers[tt + RB] = pltpu.async_copy(
                            tok_hbm.at[xc.at[tg, tt + RB]],
                            rowb.at[(tt + RB) % RB], gsem)
                    if oput[par] is not None:
                        oput[par].wait()

                    def p2body(d, c2, par=par):
                        j = d // 8
                        dd = d % 8
                        ds_ = jnp.full((L,), d, jnp.int32)
                        for m in range(BT // L):
                            bs = lax.iota(jnp.int32, L) + (m * L)
                            v = plsc.load_gather(rowp, [bs, ds_])
                            outt[par, j, dd, pl.ds(m * L, L)] = v
                        return c2

                    lax.fori_loop(0, D, p2body, 0)
                    oput[par] = pltpu.async_copy(
                        outt.at[par], o5.at[t, :, c], osem)
                for h in oput:
                    h.wait()
                return carry

            lax.fori_loop(0, ntg, tgbody, 0)

    return launch


def kernel(x, token_table, pos_table):
    B, T = x.shape
    V, D = token_table.shape
    launch = _build(B, T, D, V)
    x4 = (x.astype(jnp.int32).T
          .reshape(T // 8, 8, B // 128, 128)
          .transpose(0, 2, 1, 3))
    o5 = launch(x4, token_table, pos_table)
    return o5.transpose(2, 4, 0, 1, 3).reshape(B, T, D)


# speedup vs baseline: 1.6400x; 1.2969x over previous
import functools
import math

import jax
import jax.numpy as jnp
from jax import lax
from jax.experimental import pallas as pl
from jax.experimental.pallas import tpu as pltpu
from jax.experimental.pallas import tpu_sc as plsc


@functools.lru_cache(maxsize=None)
def _build(B, T, D, V):
    info = plsc.get_sparse_core_info()
    NC, NS, L = info.num_cores, info.num_subcores, info.num_lanes
    NW = NC * NS
    BT = 128
    TG = 8
    RB = 6
    assert B % (NW * BT) == 0 and T % TG == 0 and D % L == 0
    NBT = B // BT
    cpw = NBT // NW
    ntg = T // TG
    scale = float(math.sqrt(D))

    mesh = plsc.VectorSubcoreMesh(core_axis_name="c", subcore_axis_name="s")

    @functools.partial(
        pl.kernel,
        out_type=jax.ShapeDtypeStruct((T, D // 8, NBT, 8, BT), jnp.float32),
        mesh=mesh,
        compiler_params=pltpu.CompilerParams(use_tc_tiling_on_sc=False,
                                             needs_layout_passes=False),
        scratch_types=[
            pltpu.VMEM((T, D), jnp.float32),
            pltpu.VMEM((ntg, TG, BT), jnp.int32),
            pltpu.VMEM((RB, BT, D), jnp.float32),
            pltpu.VMEM((BT, D), jnp.float32),
            pltpu.VMEM((2, D // 8, 8, BT), jnp.float32),
            pltpu.SemaphoreType.DMA,
            pltpu.SemaphoreType.DMA,
            pltpu.SemaphoreType.DMA,
        ],
    )
    def launch(x4, tok_hbm, pos_hbm, o5, posblk, xc, rowb, rowp, outt,
               xsem, gsem, osem):
        wid = lax.axis_index("s") * NC + lax.axis_index("c")
        c0 = wid * cpw
        pltpu.sync_copy(pos_hbm.at[pl.ds(0, T)], posblk)

        for cl in range(cpw):
            c = c0 + cl
            pltpu.async_copy(x4.at[:, c], xc, xsem).wait()

            def tgbody(tg, carry):
                t0 = tg * TG
                gathers = [None] * TG
                for tt in range(RB):
                    gathers[tt] = pltpu.async_copy(
                        tok_hbm.at[xc.at[tg, tt]], rowb.at[tt % RB], gsem)
                oput = [None, None]
                for tt in range(TG):
                    t = t0 + tt
                    par = tt % 2
                    gathers[tt].wait()

                    pv = [posblk[t, pl.ds(k * L, L)] for k in range(D // L)]
                    lanes = lax.iota(jnp.int32, L)

                    def p1body(b, c2, tt=tt, pv=pv):
                        rot = (lanes - b) & (L - 1)
                        for k in range(D // L):
                            v = rowb[tt % RB, b, pl.ds(k * L, L)]
                            v = v * scale + pv[k]
                            v = v.at[rot].get(mode="promise_in_bounds")
                            rowp[b, pl.ds(k * L, L)] = v
                        return c2

                    lax.fori_loop(0, BT, p1body, 0, unroll=2)
                    if tt + RB < TG:
                        gathers[tt + RB] = pltpu.async_copy(
                            tok_hbm.at[xc.at[tg, tt + RB]],
                            rowb.at[(tt + RB) % RB], gsem)
                    if oput[par] is not None:
                        oput[par].wait()

                    def p2body(d, c2, par=par):
                        j = d // 8
                        dd = d % 8
                        col = ((d % L) + lanes) & (L - 1)
                        cols = (d // L) * L + col
                        for m in range(BT // L):
                            bs = lanes + (m * L)
                            v = plsc.load_gather(rowp, [bs, cols])
                            outt[par, j, dd, pl.ds(m * L, L)] = v
                        return c2

                    lax.fori_loop(0, D, p2body, 0)
                    oput[par] = pltpu.async_copy(
                        outt.at[par], o5.at[t, :, c], osem)
                for h in oput:
                    h.wait()
                return carry

            lax.fori_loop(0, ntg, tgbody, 0)

    return launch


def kernel(x, token_table, pos_table):
    B, T = x.shape
    V, D = token_table.shape
    launch = _build(B, T, D, V)
    x4 = (x.astype(jnp.int32).T
          .reshape(T // 8, 8, B // 128, 128)
          .transpose(0, 2, 1, 3))
    o5 = launch(x4, token_table, pos_table)
    return o5.transpose(2, 4, 0, 1, 3).reshape(B, T, D)
